# baseline (speedup 1.0000x reference)
import functools

import jax
import jax.numpy as jnp
from jax import lax
from jax.experimental import pallas as pl
from jax.experimental.pallas import tpu as pltpu
from jax.experimental.pallas import tpu_sc as plsc

NC = 2
NS = 16
LANES = 16
RB = 16


def _fuse_slice_sc(inp, starts, S, B, L, total, rows_total, per_b, n_it):
    mesh = plsc.VectorSubcoreMesh(
        core_axis_name="c", subcore_axis_name="s",
        num_cores=NC, num_subcores=NS)

    @functools.partial(
        pl.kernel,
        out_type=jax.ShapeDtypeStruct((rows_total, L), jnp.float32),
        mesh=mesh,
        scratch_types=(
            [pltpu.VMEM((S + LANES,), jnp.int32)]
            + [pltpu.VMEM((RB, total), jnp.float32)]
            + [pltpu.VMEM_SHARED((NS, RB, total), jnp.float32)]
            + [pltpu.SemaphoreType.DMA] * 4
        ),
    )
    def k(inp_hbm, starts_hbm, out_hbm, starts_v, buf_t, shared, g0, g1, s0, s1):
        sid = lax.axis_index("s")
        bufs = (buf_t, shared.at[sid])
        gsems, ssems = (g0, g1), (s0, s1)
        wid = lax.axis_index("s") * NC + lax.axis_index("c")
        pltpu.sync_copy(starts_hbm, starts_v.at[pl.ds(0, S)])
        sts = [pl.multiple_of(starts_v[pl.ds(s, LANES)][0], L) for s in range(S)]
        base = wid * per_b

        def start_gather(i, buf, gsem):
            b0 = pl.multiple_of(base + i * RB, RB)
            pltpu.async_copy(inp_hbm.at[pl.ds(b0, RB)], buf, gsem)

        def wait_gather(buf, gsem):
            pltpu.make_async_copy(inp_hbm.at[pl.ds(0, RB)], buf, gsem).wait()

        def wait_scatters(buf, ssem):
            for s in range(S):
                pltpu.make_async_copy(
                    buf.at[:, pl.ds(0, L)], out_hbm.at[pl.ds(0, RB)], ssem).wait()

        start_gather(0, bufs[0], gsems[0])
        start_gather(1, bufs[1], gsems[1])

        def phase(i, cur):
            buf, gsem, ssem = bufs[cur], gsems[cur], ssems[cur]
            b0 = pl.multiple_of(base + i * RB, RB)
            wait_gather(buf, gsem)
            for s in range(S):
                pltpu.async_copy(
                    buf.at[:, pl.ds(sts[s], L)],
                    out_hbm.at[pl.ds(s * B + b0, RB)], ssem)
            i2 = i + 2

            @pl.when(i2 < n_it)
            def _():
                wait_scatters(buf, ssem)
                start_gather(i2, buf, gsem)

        def body(j, carry):
            phase(j * 2, 0)
            phase(j * 2 + 1, 1)
            return carry

        lax.fori_loop(0, n_it // 2, body, 0)
        wait_scatters(bufs[0], ssems[0])
        wait_scatters(bufs[1], ssems[1])

    return k(inp, starts)


def kernel(input_tensor, slices_index, slice_len):
    B, total = input_tensor.shape
    S = slices_index.shape[0]
    L = total // S
    zero_offset = jnp.asarray(slice_len, jnp.int32) - jnp.int32(L)
    starts = slices_index.astype(jnp.int32) + zero_offset

    rows_total = S * B
    per_b = B // (NC * NS)
    n_it = per_b // RB
    assert per_b * NC * NS == B and n_it * RB == per_b

    out = _fuse_slice_sc(input_tensor, starts, S, B, L, total, rows_total,
                         per_b, n_it)
    return out.reshape(S, B, L)


# speedup vs baseline: 1.0212x; 1.0212x over previous
import functools

import jax
import jax.numpy as jnp
from jax import lax
from jax.experimental import pallas as pl
from jax.experimental.pallas import tpu as pltpu
from jax.experimental.pallas import tpu_sc as plsc

NC = 2
NS = 16
LANES = 16
RB = 8
NBF = 4
SLACK = 2


def _fuse_slice_sc(inp, starts, S, B, L, total, rows_total, per_b, n_it):
    mesh = plsc.VectorSubcoreMesh(
        core_axis_name="c", subcore_axis_name="s",
        num_cores=NC, num_subcores=NS)

    @functools.partial(
        pl.kernel,
        out_type=jax.ShapeDtypeStruct((rows_total, L), jnp.float32),
        mesh=mesh,
        scratch_types=(
            [pltpu.VMEM((S + LANES,), jnp.int32)]
            + [pltpu.VMEM_SHARED((NS, NBF, RB, total), jnp.float32)]
            + [pltpu.SemaphoreType.DMA] * (2 * NBF)
        ),
    )
    def k(inp_hbm, starts_hbm, out_hbm, starts_v, shared, *sems):
        gsems = sems[:NBF]
        ssems = sems[NBF:]
        sid = lax.axis_index("s")
        bufs = tuple(shared.at[sid, b] for b in range(NBF))
        wid = lax.axis_index("s") * NC + lax.axis_index("c")
        pltpu.sync_copy(starts_hbm, starts_v.at[pl.ds(0, S)])
        sts = [pl.multiple_of(starts_v[pl.ds(s, LANES)][0], L) for s in range(S)]
        base = wid * per_b

        def start_gather(i, buf, gsem):
            b0 = pl.multiple_of(base + i * RB, RB)
            pltpu.async_copy(inp_hbm.at[pl.ds(b0, RB)], buf, gsem)

        def wait_gather(buf, gsem):
            pltpu.make_async_copy(inp_hbm.at[pl.ds(0, RB)], buf, gsem).wait()

        def wait_scatters(buf, ssem):
            for _ in range(S):
                pltpu.make_async_copy(
                    buf.at[:, pl.ds(0, L)], out_hbm.at[pl.ds(0, RB)], ssem).wait()

        for b in range(NBF):
            start_gather(b, bufs[b], gsems[b])

        def phase(i, b):
            buf, gsem, ssem = bufs[b], gsems[b], ssems[b]
            b0 = pl.multiple_of(base + i * RB, RB)
            wait_gather(buf, gsem)
            for s in range(S):
                pltpu.async_copy(
                    buf.at[:, pl.ds(sts[s], L)],
                    out_hbm.at[pl.ds(s * B + b0, RB)], ssem)
            h = i + NBF - SLACK
            b2 = (b + NBF - SLACK) % NBF

            @pl.when(jnp.logical_and(h >= NBF, h < n_it))
            def _():
                wait_scatters(bufs[b2], ssems[b2])
                start_gather(h, bufs[b2], gsems[b2])

        def body(j, carry):
            for b in range(NBF):
                phase(j * NBF + b, b)
            return carry

        lax.fori_loop(0, n_it // NBF, body, 0)
        for b in range(NBF):
            wait_scatters(bufs[b], ssems[b])

    return k(inp, starts)


def kernel(input_tensor, slices_index, slice_len):
    B, total = input_tensor.shape
    S = slices_index.shape[0]
    L = total // S
    zero_offset = jnp.asarray(slice_len, jnp.int32) - jnp.int32(L)
    starts = slices_index.astype(jnp.int32) + zero_offset

    rows_total = S * B
    per_b = B // (NC * NS)
    n_it = per_b // RB
    assert per_b * NC * NS == B and n_it * RB == per_b and n_it % NBF == 0

    out = _fuse_slice_sc(input_tensor, starts, S, B, L, total, rows_total,
                         per_b, n_it)
    return out.reshape(S, B, L)


# speedup vs baseline: 1.0303x; 1.0089x over previous
import functools

import jax
import jax.numpy as jnp
from jax import lax
from jax.experimental import pallas as pl
from jax.experimental.pallas import tpu as pltpu
from jax.experimental.pallas import tpu_sc as plsc

NC = 2
NS = 16
LANES = 16
RB = 16
NBF = 2
SLACK = 1


def _fuse_slice_sc(inp, starts, S, B, L, total, rows_total, per_b, n_it):
    mesh = plsc.VectorSubcoreMesh(
        core_axis_name="c", subcore_axis_name="s",
        num_cores=NC, num_subcores=NS)

    @functools.partial(
        pl.kernel,
        out_type=jax.ShapeDtypeStruct((rows_total, L), jnp.float32),
        mesh=mesh,
        scratch_types=(
            [pltpu.VMEM((S + LANES,), jnp.int32)]
            + [pltpu.VMEM_SHARED((NS, NBF, RB, total), jnp.float32)]
            + [pltpu.SemaphoreType.DMA] * (2 * NBF)
        ),
    )
    def k(inp_hbm, starts_hbm, out_hbm, starts_v, shared, *sems):
        gsems = sems[:NBF]
        ssems = sems[NBF:]
        sid = lax.axis_index("s")
        bufs = tuple(shared.at[sid, b] for b in range(NBF))
        wid = lax.axis_index("s") * NC + lax.axis_index("c")
        pltpu.sync_copy(starts_hbm, starts_v.at[pl.ds(0, S)])
        sts = [pl.multiple_of(starts_v[pl.ds(s, LANES)][0], L) for s in range(S)]
        base = wid * per_b

        def start_gather(i, buf, gsem):
            b0 = pl.multiple_of(base + i * RB, RB)
            pltpu.async_copy(inp_hbm.at[pl.ds(b0, RB)], buf, gsem)

        def wait_gather(buf, gsem):
            pltpu.make_async_copy(inp_hbm.at[pl.ds(0, RB)], buf, gsem).wait()

        def wait_scatters(buf, ssem):
            for _ in range(S):
                pltpu.make_async_copy(
                    buf.at[:, pl.ds(0, L)], out_hbm.at[pl.ds(0, RB)], ssem).wait()

        for b in range(NBF):
            start_gather(b, bufs[b], gsems[b])

        def phase(i, b):
            buf, gsem, ssem = bufs[b], gsems[b], ssems[b]
            b0 = pl.multiple_of(base + i * RB, RB)
            wait_gather(buf, gsem)
            for s in range(S):
                pltpu.async_copy(
                    buf.at[:, pl.ds(sts[s], L)],
                    out_hbm.at[pl.ds(s * B + b0, RB)], ssem)
            h = i + NBF - SLACK
            b2 = (b + NBF - SLACK) % NBF

            @pl.when(jnp.logical_and(h >= NBF, h < n_it))
            def _():
                wait_scatters(bufs[b2], ssems[b2])
                start_gather(h, bufs[b2], gsems[b2])

        def body(j, carry):
            for b in range(NBF):
                phase(j * NBF + b, b)
            return carry

        lax.fori_loop(0, n_it // NBF, body, 0)
        for b in range(NBF):
            wait_scatters(bufs[b], ssems[b])

    return k(inp, starts)


def kernel(input_tensor, slices_index, slice_len):
    B, total = input_tensor.shape
    S = slices_index.shape[0]
    L = total // S
    zero_offset = jnp.asarray(slice_len, jnp.int32) - jnp.int32(L)
    starts = slices_index.astype(jnp.int32) + zero_offset

    rows_total = S * B
    per_b = B // (NC * NS)
    n_it = per_b // RB
    assert per_b * NC * NS == B and n_it * RB == per_b and n_it % NBF == 0

    out = _fuse_slice_sc(input_tensor, starts, S, B, L, total, rows_total,
                         per_b, n_it)
    return out.reshape(S, B, L)


# speedup vs baseline: 1.0346x; 1.0042x over previous
import functools

import jax
import jax.numpy as jnp
from jax import lax
from jax.experimental import pallas as pl
from jax.experimental.pallas import tpu as pltpu
from jax.experimental.pallas import tpu_sc as plsc

NC = 2
NS = 16
LANES = 16
RB = 16


def _fuse_slice_sc(inp, starts, S, B, L, total, rows_total, per_b, n_it):
    mesh = plsc.VectorSubcoreMesh(
        core_axis_name="c", subcore_axis_name="s",
        num_cores=NC, num_subcores=NS)

    @functools.partial(
        pl.kernel,
        out_type=jax.ShapeDtypeStruct((rows_total, L), jnp.float32),
        mesh=mesh,
        scratch_types=(
            [pltpu.VMEM((S + LANES,), jnp.int32)]
            + [pltpu.VMEM_SHARED((NS, 2, RB, total), jnp.float32)]
            + [pltpu.SemaphoreType.DMA] * 4
        ),
    )
    def k(inp_hbm, starts_hbm, out_hbm, starts_v, shared, g0, g1, s0, s1):
        sid = lax.axis_index("s")
        bufs = (shared.at[sid, 0], shared.at[sid, 1])
        gsems, ssems = (g0, g1), (s0, s1)
        wid = lax.axis_index("s") * NC + lax.axis_index("c")
        pltpu.sync_copy(starts_hbm, starts_v.at[pl.ds(0, S)])
        sts = [pl.multiple_of(starts_v[pl.ds(s, LANES)][0], L) for s in range(S)]
        base = wid * per_b

        def start_gather(i, buf, gsem):
            b0 = pl.multiple_of(base + i * RB, RB)
            pltpu.async_copy(inp_hbm.at[pl.ds(b0, RB)], buf, gsem)

        def wait_gather(buf, gsem):
            pltpu.make_async_copy(inp_hbm.at[pl.ds(0, RB)], buf, gsem).wait()

        def wait_scatters(ssem):
            pltpu.make_async_copy(
                out_hbm.at[pl.ds(0, RB * S)], out_hbm.at[pl.ds(0, RB * S)],
                ssem).wait()

        start_gather(0, bufs[0], gsems[0])
        start_gather(1, bufs[1], gsems[1])

        def phase(i, cur):
            buf, gsem, ssem = bufs[cur], gsems[cur], ssems[cur]
            b0 = pl.multiple_of(base + i * RB, RB)
            wait_gather(buf, gsem)
            for s in range(S):
                pltpu.async_copy(
                    buf.at[:, pl.ds(sts[s], L)],
                    out_hbm.at[pl.ds(s * B + b0, RB)], ssem)
            h = i + 1
            oth = 1 - cur

            @pl.when(jnp.logical_and(h >= 2, h < n_it))
            def _():
                wait_scatters(ssems[oth])
                start_gather(h, bufs[oth], gsems[oth])

        def body(j, carry):
            phase(j * 2, 0)
            phase(j * 2 + 1, 1)
            return carry

        lax.fori_loop(0, n_it // 2, body, 0)
        wait_scatters(ssems[0])
        wait_scatters(ssems[1])

    return k(inp, starts)


def kernel(input_tensor, slices_index, slice_len):
    B, total = input_tensor.shape
    S = slices_index.shape[0]
    L = total // S
    zero_offset = jnp.asarray(slice_len, jnp.int32) - jnp.int32(L)
    starts = slices_index.astype(jnp.int32) + zero_offset

    rows_total = S * B
    per_b = B // (NC * NS)
    n_it = per_b // RB
    assert per_b * NC * NS == B and n_it * RB == per_b and n_it % 2 == 0

    out = _fuse_slice_sc(input_tensor, starts, S, B, L, total, rows_total,
                         per_b, n_it)
    return out.reshape(S, B, L)
